# baseline (speedup 1.0000x reference)
import functools

import jax
import jax.numpy as jnp
from jax import lax
from jax.experimental import pallas as pl
from jax.experimental.pallas import tpu as pltpu
from jax.experimental.pallas import tpu_sc as plsc

F32 = jnp.float32
I32 = jnp.int32

N_BOXES = 5000
NPAD = 5120
BLK = 512
NBLK = NPAD // BLK
NMS_THR = 0.5
SCORE_THR = 0.2

N_PAIRS = 512
POOL_DIM = 12544
REP = 1024
NUM_CLASSES = 117
CPAD = 128
K_TILE = 1792
K_STEPS = POOL_DIM // K_TILE


def _iou_gt(bo, boT, s, t):
    rx1 = bo[s * BLK:(s + 1) * BLK, 0:1]
    ry1 = bo[s * BLK:(s + 1) * BLK, 1:2]
    rx2 = bo[s * BLK:(s + 1) * BLK, 2:3]
    ry2 = bo[s * BLK:(s + 1) * BLK, 3:4]
    cx1 = boT[0:1, t * BLK:(t + 1) * BLK]
    cy1 = boT[1:2, t * BLK:(t + 1) * BLK]
    cx2 = boT[2:3, t * BLK:(t + 1) * BLK]
    cy2 = boT[3:4, t * BLK:(t + 1) * BLK]
    ix1 = jnp.maximum(rx1, cx1)
    iy1 = jnp.maximum(ry1, cy1)
    ix2 = jnp.minimum(rx2, cx2)
    iy2 = jnp.minimum(ry2, cy2)
    inter = jnp.clip(ix2 - ix1, 0.0) * jnp.clip(iy2 - iy1, 0.0)
    ar = (rx2 - rx1) * (ry2 - ry1)
    ac = (cx2 - cx1) * (cy2 - cy1)
    iou = inter / (ar + ac - inter + 1e-8)
    return (iou > NMS_THR).astype(F32)


def _nms_body(bs_ref, bt_ref, lsc_ref, lsr_ref, li_ref, ss_ref, keep_ref,
              acc_ref):
    mc = jnp.max(bs_ref[...]) + 1.0
    bo = bs_ref[...] + lsc_ref[...] * mc
    boT = bt_ref[...] + lsr_ref[...] * mc

    ii = lax.broadcasted_iota(I32, (BLK, BLK), 0)
    jj = lax.broadcasted_iota(I32, (BLK, BLK), 1)
    upper = (ii < jj).astype(F32)

    lmin = [jnp.min(li_ref[0, t * BLK:(t + 1) * BLK]) for t in range(NBLK)]
    lmax = [jnp.max(li_ref[0, t * BLK:(t + 1) * BLK]) for t in range(NBLK)]

    for t in range(NBLK):
        cand0 = (ss_ref[0:1, t * BLK:(t + 1) * BLK] > SCORE_THR).astype(F32)
        acc_ref[...] = jnp.zeros((1, BLK), F32)
        for s in range(t):
            @pl.when(lmax[s] == lmin[t])
            def _(s=s, t=t):
                m = _iou_gt(bo, boT, s, t)
                ks = keep_ref[0:1, s * BLK:(s + 1) * BLK]
                acc_ref[...] += jnp.dot(ks, m, preferred_element_type=F32)
        cand = cand0 * (acc_ref[...] == 0.0).astype(F32)
        m_in = _iou_gt(bo, boT, t, t) * upper

        def step(alive):
            supp = jnp.dot(alive, m_in, preferred_element_type=F32)
            return cand * (supp == 0.0).astype(F32)

        def cond(st):
            alive, prev, it = st
            return jnp.logical_and(jnp.any(alive != prev), it < BLK)

        def body(st):
            alive, prev, it = st
            return step(alive), alive, it + 1

        alive, _, _ = lax.while_loop(cond, body, (step(cand), cand, 1))
        keep_ref[0:1, t * BLK:(t + 1) * BLK] = alive


def _nms_keep_sorted(bs, bt, ls_col, ls_row, li_row, ss_row):
    return pl.pallas_call(
        _nms_body,
        out_shape=jax.ShapeDtypeStruct((1, NPAD), F32),
        scratch_shapes=[pltpu.VMEM((1, BLK), F32)],
    )(bs, bt, ls_col, ls_row, li_row, ss_row)


_SC_NC = 2
_SC_NS = 16
_SC_LANES = 16


def _sc_prior_body(keep_hbm, order_hbm, scores_hbm, labels_hbm, o2t_hbm,
                   h_hbm, o_hbm, out_hbm,
                   keep_v, order_v, korig_v, scores_v, labels_v, o2t_v,
                   h_v, o_v, out_v):
    wid = lax.axis_index("s") * _SC_NC + lax.axis_index("c")

    pltpu.sync_copy(keep_hbm, keep_v)
    pltpu.sync_copy(order_hbm, order_v)
    pltpu.sync_copy(scores_hbm, scores_v)
    pltpu.sync_copy(labels_hbm, labels_v)
    pltpu.sync_copy(o2t_hbm, o2t_v)
    pltpu.sync_copy(h_hbm.at[pl.ds(wid * _SC_LANES, _SC_LANES)], h_v)
    pltpu.sync_copy(o_hbm.at[pl.ds(wid * _SC_LANES, _SC_LANES)], o_v)

    def scatter_body(i, carry):
        kv = keep_v[pl.ds(i * _SC_LANES, _SC_LANES)]
        ov = order_v[pl.ds(i * _SC_LANES, _SC_LANES)]
        plsc.store_scatter(korig_v, [ov], kv)
        return carry

    lax.fori_loop(0, NPAD // _SC_LANES, scatter_body, 0)

    hv = h_v[...]
    ov = o_v[...]
    sh = plsc.load_gather(scores_v, [hv])
    so = plsc.load_gather(scores_v, [ov])
    kh = plsc.load_gather(korig_v, [hv])
    ko = plsc.load_gather(korig_v, [ov])
    lo = plsc.load_gather(labels_v, [ov])
    ne = (hv != ov).astype(F32)
    w = sh * so * kh * ko * ne
    base = lo * CPAD
    lanes = lax.iota(I32, _SC_LANES) * CPAD

    def col_body(c, carry):
        col = plsc.load_gather(o2t_v, [base + c])
        plsc.store_scatter(out_v, [lanes + c], w * col)
        return carry

    lax.fori_loop(0, CPAD, col_body, 0)
    pltpu.sync_copy(out_v, out_hbm.at[pl.ds(wid * _SC_LANES * CPAD,
                                            _SC_LANES * CPAD)])


@functools.partial(
    pl.kernel,
    mesh=plsc.VectorSubcoreMesh(core_axis_name="c", subcore_axis_name="s"),
    out_type=jax.ShapeDtypeStruct((N_PAIRS * CPAD,), F32),
    compiler_params=pltpu.CompilerParams(needs_layout_passes=False),
    scratch_types=[
        pltpu.VMEM((NPAD,), F32),
        pltpu.VMEM((NPAD,), I32),
        pltpu.VMEM((NPAD,), F32),
        pltpu.VMEM((NPAD,), F32),
        pltpu.VMEM((NPAD,), I32),
        pltpu.VMEM((80 * CPAD,), F32),
        pltpu.VMEM((_SC_LANES,), I32),
        pltpu.VMEM((_SC_LANES,), I32),
        pltpu.VMEM((_SC_LANES * CPAD,), F32),
    ],
)
def _sc_prior(*args):
    _sc_prior_body(*args)


def _mlp1_body(pf_ref, w1_ref, b1_ref, out_ref):
    k = pl.program_id(0)

    @pl.when(k == 0)
    def _():
        out_ref[...] = jnp.broadcast_to(b1_ref[...], (N_PAIRS, REP))

    out_ref[...] += jnp.dot(pf_ref[...], w1_ref[...],
                            preferred_element_type=F32)

    @pl.when(k == K_STEPS - 1)
    def _():
        out_ref[...] = jnp.maximum(out_ref[...], 0.0)


def _mlp1(pf_bf16, W1_bf16, b1_row):
    return pl.pallas_call(
        _mlp1_body,
        grid=(K_STEPS,),
        in_specs=[
            pl.BlockSpec((N_PAIRS, K_TILE), lambda k: (0, k)),
            pl.BlockSpec((K_TILE, REP), lambda k: (k, 0)),
            pl.BlockSpec((1, REP), lambda k: (0, 0)),
        ],
        out_specs=pl.BlockSpec((N_PAIRS, REP), lambda k: (0, 0)),
        out_shape=jax.ShapeDtypeStruct((N_PAIRS, REP), F32),
    )(pf_bf16, W1_bf16, b1_row)


def _mlp2_body(x1_ref, w2_ref, b2_ref, w3_ref, b3_ref, prior_ref, out_ref):
    x = jnp.dot(x1_ref[...], w2_ref[...], preferred_element_type=F32)
    x = jnp.maximum(x + b2_ref[...], 0.0)
    logits = jnp.dot(x, w3_ref[...], preferred_element_type=F32) + b3_ref[...]
    sig = 1.0 / (1.0 + jnp.exp(-logits))
    out_ref[...] = sig * prior_ref[...]


def _mlp2(x1, W2, b2_row, W3p, b3p_row, prior):
    return pl.pallas_call(
        _mlp2_body,
        out_shape=jax.ShapeDtypeStruct((N_PAIRS, CPAD), F32),
    )(x1, W2, b2_row, W3p, b3p_row, prior)


def kernel(boxes, scores, pair_features, W1, b1, W2, b2, W3, b3, obj2tgt,
           labels, paired_idx):
    labels32 = labels.astype(I32)
    h = paired_idx[:, 0].astype(I32)
    o = paired_idx[:, 1].astype(I32)

    pad = NPAD - N_BOXES
    scores_p = jnp.concatenate([scores, jnp.full((pad,), -1.0, F32)])
    boxes_p = jnp.concatenate([boxes, jnp.broadcast_to(boxes[0:1], (pad, 4))])
    labels_p = jnp.concatenate([labels32, jnp.full((pad,), 80, I32)])
    iota = jnp.arange(NPAD, dtype=I32)

    lab_s, _, order, bx1, by1, bx2, by2, ss = lax.sort(
        (labels_p, -scores_p, iota, boxes_p[:, 0], boxes_p[:, 1],
         boxes_p[:, 2], boxes_p[:, 3], scores_p),
        num_keys=2, is_stable=True)

    bs = jnp.stack([bx1, by1, bx2, by2], axis=1)
    bt = jnp.stack([bx1, by1, bx2, by2], axis=0)
    ls_col = lab_s.astype(F32)[:, None]
    ls_row = lab_s.astype(F32)[None, :]
    li_row = lab_s[None, :]
    ss_row = ss[None, :]

    keep_row = _nms_keep_sorted(bs, bt, ls_col, ls_row, li_row, ss_row)
    keep_sorted = keep_row[0]

    o2t_flat = jnp.pad(obj2tgt, ((0, 0), (0, CPAD - NUM_CLASSES))).reshape(-1)
    prior = _sc_prior(keep_sorted, order, scores_p, labels_p, o2t_flat,
                      h, o).reshape(N_PAIRS, CPAD)

    x1 = _mlp1(pair_features.astype(jnp.bfloat16), W1.astype(jnp.bfloat16),
               b1[None, :])
    W3p = jnp.pad(W3, ((0, 0), (0, CPAD - NUM_CLASSES)))
    b3p = jnp.pad(b3, (0, CPAD - NUM_CLASSES))
    out_pad = _mlp2(x1, W2, b2[None, :], W3p, b3p[None, :], prior)
    return out_pad[:, :NUM_CLASSES]


# speedup vs baseline: 1.3152x; 1.3152x over previous
import functools

import jax
import jax.numpy as jnp
from jax import lax
from jax.experimental import pallas as pl
from jax.experimental.pallas import tpu as pltpu
from jax.experimental.pallas import tpu_sc as plsc

F32 = jnp.float32
I32 = jnp.int32

N_BOXES = 5000
NPAD = 5120
BLK = 512
NBLK = NPAD // BLK
NMS_THR = 0.5
SCORE_THR = 0.2

N_PAIRS = 512
POOL_DIM = 12544
REP = 1024
NUM_CLASSES = 117
CPAD = 128
K_TILE = 1792
K_STEPS = POOL_DIM // K_TILE


def _iou_gt(bo, boT, s, t):
    rx1 = bo[s * BLK:(s + 1) * BLK, 0:1]
    ry1 = bo[s * BLK:(s + 1) * BLK, 1:2]
    rx2 = bo[s * BLK:(s + 1) * BLK, 2:3]
    ry2 = bo[s * BLK:(s + 1) * BLK, 3:4]
    cx1 = boT[0:1, t * BLK:(t + 1) * BLK]
    cy1 = boT[1:2, t * BLK:(t + 1) * BLK]
    cx2 = boT[2:3, t * BLK:(t + 1) * BLK]
    cy2 = boT[3:4, t * BLK:(t + 1) * BLK]
    ix1 = jnp.maximum(rx1, cx1)
    iy1 = jnp.maximum(ry1, cy1)
    ix2 = jnp.minimum(rx2, cx2)
    iy2 = jnp.minimum(ry2, cy2)
    inter = jnp.clip(ix2 - ix1, 0.0) * jnp.clip(iy2 - iy1, 0.0)
    ar = (rx2 - rx1) * (ry2 - ry1)
    ac = (cx2 - cx1) * (cy2 - cy1)
    iou = inter / (ar + ac - inter + 1e-8)
    return (iou > NMS_THR).astype(F32)


def _nms_body(bs_ref, bt_ref, lsc_ref, lsr_ref, li_ref, ss_ref, keep_ref,
              acc_ref):
    mc = jnp.max(bs_ref[...]) + 1.0
    bo = bs_ref[...] + lsc_ref[...] * mc
    boT = bt_ref[...] + lsr_ref[...] * mc

    ii = lax.broadcasted_iota(I32, (BLK, BLK), 0)
    jj = lax.broadcasted_iota(I32, (BLK, BLK), 1)
    upper = (ii < jj).astype(F32)

    lmin = [jnp.min(li_ref[0, t * BLK:(t + 1) * BLK]) for t in range(NBLK)]
    lmax = [jnp.max(li_ref[0, t * BLK:(t + 1) * BLK]) for t in range(NBLK)]

    for t in range(NBLK):
        cand0 = (ss_ref[0:1, t * BLK:(t + 1) * BLK] > SCORE_THR).astype(F32)
        acc_ref[...] = jnp.zeros((1, BLK), F32)
        for s in range(t):
            @pl.when(lmax[s] == lmin[t])
            def _(s=s, t=t):
                m = _iou_gt(bo, boT, s, t)
                ks = keep_ref[0:1, s * BLK:(s + 1) * BLK]
                acc_ref[...] += jnp.dot(ks, m, preferred_element_type=F32)
        cand = cand0 * (acc_ref[...] == 0.0).astype(F32)
        m_in = _iou_gt(bo, boT, t, t) * upper

        def step(alive):
            supp = jnp.dot(alive, m_in, preferred_element_type=F32)
            return cand * (supp == 0.0).astype(F32)

        def cond(st):
            alive, prev, it = st
            return jnp.logical_and(jnp.any(alive != prev), it < BLK)

        def body(st):
            alive, prev, it = st
            return step(alive), alive, it + 1

        alive, _, _ = lax.while_loop(cond, body, (step(cand), cand, 1))
        keep_ref[0:1, t * BLK:(t + 1) * BLK] = alive


def _nms_keep_sorted(bs, bt, ls_col, ls_row, li_row, ss_row):
    return pl.pallas_call(
        _nms_body,
        out_shape=jax.ShapeDtypeStruct((1, NPAD), F32),
        scratch_shapes=[pltpu.VMEM((1, BLK), F32)],
    )(bs, bt, ls_col, ls_row, li_row, ss_row)


_SC_NC = 2
_SC_NS = 16
_SC_LANES = 16


def _sc_prior_body(keep_hbm, order_hbm, scores_hbm, labels_hbm, o2t_hbm,
                   h_hbm, o_hbm, out_hbm,
                   keep_v, order_v, korig_v, scores_v, labels_v, o2t_v,
                   h_v, o_v, out_v):
    wid = lax.axis_index("s") * _SC_NC + lax.axis_index("c")

    pltpu.sync_copy(keep_hbm, keep_v)
    pltpu.sync_copy(order_hbm, order_v)
    pltpu.sync_copy(scores_hbm, scores_v)
    pltpu.sync_copy(labels_hbm, labels_v)
    pltpu.sync_copy(o2t_hbm, o2t_v)
    pltpu.sync_copy(h_hbm.at[pl.ds(wid * _SC_LANES, _SC_LANES)], h_v)
    pltpu.sync_copy(o_hbm.at[pl.ds(wid * _SC_LANES, _SC_LANES)], o_v)

    def scatter_body(i, carry):
        kv = keep_v[pl.ds(i * _SC_LANES, _SC_LANES)]
        ov = order_v[pl.ds(i * _SC_LANES, _SC_LANES)]
        plsc.store_scatter(korig_v, [ov], kv)
        return carry

    lax.fori_loop(0, NPAD // _SC_LANES, scatter_body, 0)

    hv = h_v[...]
    ov = o_v[...]
    sh = plsc.load_gather(scores_v, [hv])
    so = plsc.load_gather(scores_v, [ov])
    kh = plsc.load_gather(korig_v, [hv])
    ko = plsc.load_gather(korig_v, [ov])
    lo = plsc.load_gather(labels_v, [ov])
    ne = (hv != ov).astype(F32)
    w = sh * so * kh * ko * ne
    base = lo * CPAD
    lanes = lax.iota(I32, _SC_LANES) * CPAD

    def col_body(c, carry):
        col = plsc.load_gather(o2t_v, [base + c])
        plsc.store_scatter(out_v, [lanes + c], w * col)
        return carry

    lax.fori_loop(0, CPAD, col_body, 0)
    pltpu.sync_copy(out_v, out_hbm.at[pl.ds(wid * _SC_LANES * CPAD,
                                            _SC_LANES * CPAD)])


@functools.partial(
    pl.kernel,
    mesh=plsc.VectorSubcoreMesh(core_axis_name="c", subcore_axis_name="s"),
    out_type=jax.ShapeDtypeStruct((N_PAIRS * CPAD,), F32),
    compiler_params=pltpu.CompilerParams(needs_layout_passes=False),
    scratch_types=[
        pltpu.VMEM((NPAD,), F32),
        pltpu.VMEM((NPAD,), I32),
        pltpu.VMEM((NPAD,), F32),
        pltpu.VMEM((NPAD,), F32),
        pltpu.VMEM((NPAD,), I32),
        pltpu.VMEM((80 * CPAD,), F32),
        pltpu.VMEM((_SC_LANES,), I32),
        pltpu.VMEM((_SC_LANES,), I32),
        pltpu.VMEM((_SC_LANES * CPAD,), F32),
    ],
)
def _sc_prior(*args):
    _sc_prior_body(*args)


def _mlp1_body(pf_ref, w1_ref, b1_ref, out_ref):
    k = pl.program_id(0)

    @pl.when(k == 0)
    def _():
        out_ref[...] = jnp.broadcast_to(b1_ref[...], (N_PAIRS, REP))

    out_ref[...] += jnp.dot(pf_ref[...].astype(jnp.bfloat16),
                            w1_ref[...].astype(jnp.bfloat16),
                            preferred_element_type=F32)

    @pl.when(k == K_STEPS - 1)
    def _():
        out_ref[...] = jnp.maximum(out_ref[...], 0.0)


def _mlp1(pf_bf16, W1_bf16, b1_row):
    return pl.pallas_call(
        _mlp1_body,
        grid=(K_STEPS,),
        in_specs=[
            pl.BlockSpec((N_PAIRS, K_TILE), lambda k: (0, k)),
            pl.BlockSpec((K_TILE, REP), lambda k: (k, 0)),
            pl.BlockSpec((1, REP), lambda k: (0, 0)),
        ],
        out_specs=pl.BlockSpec((N_PAIRS, REP), lambda k: (0, 0)),
        out_shape=jax.ShapeDtypeStruct((N_PAIRS, REP), F32),
    )(pf_bf16, W1_bf16, b1_row)


def _mlp2_body(x1_ref, w2_ref, b2_ref, w3_ref, b3_ref, prior_ref, out_ref):
    x = jnp.dot(x1_ref[...], w2_ref[...], preferred_element_type=F32)
    x = jnp.maximum(x + b2_ref[...], 0.0)
    logits = jnp.dot(x, w3_ref[...], preferred_element_type=F32) + b3_ref[...]
    sig = 1.0 / (1.0 + jnp.exp(-logits))
    out_ref[...] = sig * prior_ref[...]


def _mlp2(x1, W2, b2_row, W3p, b3p_row, prior):
    return pl.pallas_call(
        _mlp2_body,
        out_shape=jax.ShapeDtypeStruct((N_PAIRS, CPAD), F32),
    )(x1, W2, b2_row, W3p, b3p_row, prior)


def kernel(boxes, scores, pair_features, W1, b1, W2, b2, W3, b3, obj2tgt,
           labels, paired_idx):
    labels32 = labels.astype(I32)
    h = paired_idx[:, 0].astype(I32)
    o = paired_idx[:, 1].astype(I32)

    pad = NPAD - N_BOXES
    scores_p = jnp.concatenate([scores, jnp.full((pad,), -1.0, F32)])
    boxes_p = jnp.concatenate([boxes, jnp.broadcast_to(boxes[0:1], (pad, 4))])
    labels_p = jnp.concatenate([labels32, jnp.full((pad,), 80, I32)])
    iota = jnp.arange(NPAD, dtype=I32)

    lab_s, _, order, bx1, by1, bx2, by2, ss = lax.sort(
        (labels_p, -scores_p, iota, boxes_p[:, 0], boxes_p[:, 1],
         boxes_p[:, 2], boxes_p[:, 3], scores_p),
        num_keys=2, is_stable=True)

    bs = jnp.stack([bx1, by1, bx2, by2], axis=1)
    bt = jnp.stack([bx1, by1, bx2, by2], axis=0)
    ls_col = lab_s.astype(F32)[:, None]
    ls_row = lab_s.astype(F32)[None, :]
    li_row = lab_s[None, :]
    ss_row = ss[None, :]

    keep_row = _nms_keep_sorted(bs, bt, ls_col, ls_row, li_row, ss_row)
    keep_sorted = keep_row[0]

    o2t_flat = jnp.pad(obj2tgt, ((0, 0), (0, CPAD - NUM_CLASSES))).reshape(-1)
    prior = _sc_prior(keep_sorted, order, scores_p, labels_p, o2t_flat,
                      h, o).reshape(N_PAIRS, CPAD)

    x1 = _mlp1(pair_features, W1, b1[None, :])
    W3p = jnp.pad(W3, ((0, 0), (0, CPAD - NUM_CLASSES)))
    b3p = jnp.pad(b3, (0, CPAD - NUM_CLASSES))
    out_pad = _mlp2(x1, W2, b2[None, :], W3p, b3p[None, :], prior)
    return out_pad[:, :NUM_CLASSES]
